# baseline (speedup 1.0000x reference)
import jax
import jax.numpy as jnp
from jax.experimental import pallas as pl


def _fps(xyz, number):
    N = xyz.shape[0]

    def body(i, state):
        idxs, dists, farthest = state
        idxs = idxs.at[i].set(farthest)
        centroid = xyz[farthest]
        d = jnp.sum((xyz - centroid) ** 2, axis=-1)
        dists = jnp.minimum(dists, d)
        farthest = jnp.argmax(dists).astype(jnp.int32)
        return (idxs, dists, farthest)

    idxs0 = jnp.zeros((number,), dtype=jnp.int32)
    dists0 = jnp.full((N,), 1e10, dtype=jnp.float32)
    idxs, _, _ = jax.lax.fori_loop(0, number, body, (idxs0, dists0, jnp.array(0, jnp.int32)))
    return idxs


def _identity_kernel(x_ref, o_ref):
    o_ref[...] = x_ref[...]


def kernel(pc_fts, num_groups, group_size):
    B, N, C = pc_fts.shape
    G = 512
    M = 32
    xyz = pc_fts[..., :3]

    fps_idx = jax.vmap(lambda p: _fps(p, G))(xyz)
    centers = jnp.take_along_axis(
        xyz, jnp.broadcast_to(fps_idx[..., None].astype(jnp.int32), (B, G, 3)), axis=1
    )

    c2 = jnp.sum(centers ** 2, axis=-1)
    x2 = jnp.sum(xyz ** 2, axis=-1)
    cxt = jnp.einsum("bgd,bnd->bgn", centers, xyz)
    d2 = c2[:, :, None] + x2[:, None, :] - 2.0 * cxt
    _, idx = jax.lax.top_k(-d2, M)

    idx_base = jnp.arange(B, dtype=idx.dtype)[:, None, None] * N
    flat_idx = (idx + idx_base).reshape(-1)
    flat = pc_fts.reshape(B * N, C)
    neighborhoods = jnp.take(flat, flat_idx, axis=0).reshape(B, G, M, C)
    neighborhoods = neighborhoods.at[..., :3].add(-centers[:, :, None, :])

    flat2 = neighborhoods.reshape(1536, 1024)
    flat2 = pl.pallas_call(
        _identity_kernel,
        out_shape=jax.ShapeDtypeStruct(flat2.shape, flat2.dtype),
    )(flat2)
    neighborhoods = flat2.reshape(B, G, M, C)
    return (neighborhoods, centers)


# speedup vs baseline: 1.6051x; 1.6051x over previous
import functools

import jax
import jax.numpy as jnp
from jax.experimental import pallas as pl
from jax.experimental.pallas import tpu as pltpu


def _fps_body(x_ref, y_ref, z_ref, cx_ref, cy_ref, cz_ref, dists_ref, niota_ref, *, n_groups):
    B, N = x_ref.shape
    dists_ref[...] = jnp.full((B, N), 1e10, dtype=jnp.float32)
    niota_ref[...] = jax.lax.broadcasted_iota(jnp.int32, (B, N), 1)

    x = x_ref[...]
    y = y_ref[...]
    z = z_ref[...]
    ni = niota_ref[...]

    cx0 = x[:, 0:1]
    cy0 = y[:, 0:1]
    cz0 = z[:, 0:1]

    lane_i = jax.lax.broadcasted_iota(jnp.int32, (B, 128), 1)
    zeros_blk = jnp.zeros((B, 128), dtype=jnp.float32)

    def body(i, carry):
        cx, cy, cz, bx, by, bz = carry
        msk = lane_i == i
        bx = jnp.where(msk, cx, bx)
        by = jnp.where(msk, cy, by)
        bz = jnp.where(msk, cz, bz)
        dx = x - cx
        dy = y - cy
        dz = z - cz
        d = dx * dx + dy * dy + dz * dz
        dn = jnp.minimum(dists_ref[...], d)
        dists_ref[...] = dn
        m = jnp.max(dn, axis=1, keepdims=True)
        am = jnp.min(jnp.where(dn == m, ni, jnp.int32(1 << 30)), axis=1, keepdims=True)
        sel = ni == am
        ncx = jnp.max(jnp.where(sel, x, -jnp.inf), axis=1, keepdims=True)
        ncy = jnp.max(jnp.where(sel, y, -jnp.inf), axis=1, keepdims=True)
        ncz = jnp.max(jnp.where(sel, z, -jnp.inf), axis=1, keepdims=True)
        return (ncx, ncy, ncz, bx, by, bz)

    carry = (cx0, cy0, cz0, zeros_blk, zeros_blk, zeros_blk)
    for blk in range(n_groups // 128):
        carry = jax.lax.fori_loop(0, 128, body, carry)
        cx_ref[:, blk * 128:(blk + 1) * 128] = carry[3]
        cy_ref[:, blk * 128:(blk + 1) * 128] = carry[4]
        cz_ref[:, blk * 128:(blk + 1) * 128] = carry[5]


def _fps_centers(xyz, n_groups):
    B, N, _ = xyz.shape
    x = xyz[..., 0]
    y = xyz[..., 1]
    z = xyz[..., 2]
    out_sh = jax.ShapeDtypeStruct((B, n_groups), jnp.float32)
    cx, cy, cz = pl.pallas_call(
        functools.partial(_fps_body, n_groups=n_groups),
        out_shape=(out_sh, out_sh, out_sh),
        scratch_shapes=[
            pltpu.VMEM((B, N), jnp.float32),
            pltpu.VMEM((B, N), jnp.int32),
        ],
    )(x, y, z)
    return jnp.stack([cx, cy, cz], axis=-1)


def kernel(pc_fts, num_groups, group_size):
    B, N, C = pc_fts.shape
    G = 512
    M = 32
    xyz = pc_fts[..., :3]

    centers = _fps_centers(xyz, G)

    c2 = jnp.sum(centers ** 2, axis=-1)
    x2 = jnp.sum(xyz ** 2, axis=-1)
    cxt = jnp.einsum("bgd,bnd->bgn", centers, xyz)
    d2 = c2[:, :, None] + x2[:, None, :] - 2.0 * cxt
    _, idx = jax.lax.top_k(-d2, M)

    idx_base = jnp.arange(B, dtype=idx.dtype)[:, None, None] * N
    flat_idx = (idx + idx_base).reshape(-1)
    flat = pc_fts.reshape(B * N, C)
    neighborhoods = jnp.take(flat, flat_idx, axis=0).reshape(B, G, M, C)
    neighborhoods = neighborhoods.at[..., :3].add(-centers[:, :, None, :])
    return (neighborhoods, centers)


# speedup vs baseline: 1.8760x; 1.1688x over previous
import functools

import jax
import jax.numpy as jnp
from jax.experimental import pallas as pl
from jax.experimental.pallas import tpu as pltpu
from jax.experimental.pallas import tpu_sc as plsc


def _fps_body(x_ref, y_ref, z_ref, cx_ref, cy_ref, cz_ref, dists_ref, niota_ref, *, n_groups):
    B, N = x_ref.shape
    dists_ref[...] = jnp.full((B, N), 1e10, dtype=jnp.float32)
    niota_ref[...] = jax.lax.broadcasted_iota(jnp.int32, (B, N), 1)

    x = x_ref[...]
    y = y_ref[...]
    z = z_ref[...]
    ni = niota_ref[...]

    cx0 = x[:, 0:1]
    cy0 = y[:, 0:1]
    cz0 = z[:, 0:1]

    lane_i = jax.lax.broadcasted_iota(jnp.int32, (B, 128), 1)
    zeros_blk = jnp.zeros((B, 128), dtype=jnp.float32)

    def body(i, carry):
        cx, cy, cz, bx, by, bz = carry
        msk = lane_i == i
        bx = jnp.where(msk, cx, bx)
        by = jnp.where(msk, cy, by)
        bz = jnp.where(msk, cz, bz)
        dx = x - cx
        dy = y - cy
        dz = z - cz
        d = (dx * dx + dz * dz) + dy * dy
        dn = jnp.minimum(dists_ref[...], d)
        dists_ref[...] = dn
        m = jnp.max(dn, axis=1, keepdims=True)
        am = jnp.min(jnp.where(dn == m, ni, jnp.int32(1 << 30)), axis=1, keepdims=True)
        sel = ni == am
        ncx = jnp.max(jnp.where(sel, x, -jnp.inf), axis=1, keepdims=True)
        ncy = jnp.max(jnp.where(sel, y, -jnp.inf), axis=1, keepdims=True)
        ncz = jnp.max(jnp.where(sel, z, -jnp.inf), axis=1, keepdims=True)
        return (ncx, ncy, ncz, bx, by, bz)

    carry = (cx0, cy0, cz0, zeros_blk, zeros_blk, zeros_blk)
    for blk in range(n_groups // 128):
        carry = jax.lax.fori_loop(0, 128, body, carry)
        cx_ref[:, blk * 128:(blk + 1) * 128] = carry[3]
        cy_ref[:, blk * 128:(blk + 1) * 128] = carry[4]
        cz_ref[:, blk * 128:(blk + 1) * 128] = carry[5]


def _fps_centers(xyz, n_groups):
    B, N, _ = xyz.shape
    x = xyz[..., 0]
    y = xyz[..., 1]
    z = xyz[..., 2]
    out_sh = jax.ShapeDtypeStruct((B, n_groups), jnp.float32)
    cx, cy, cz = pl.pallas_call(
        functools.partial(_fps_body, n_groups=n_groups),
        out_shape=(out_sh, out_sh, out_sh),
        scratch_shapes=[
            pltpu.VMEM((B, N), jnp.float32),
            pltpu.VMEM((B, N), jnp.int32),
        ],
    )(x, y, z)
    return jnp.stack([cx, cy, cz], axis=-1)


_CHUNK = 128


def _sc_gather_rows(table128, row_idx):
    K = row_idx.shape[0]
    mesh = plsc.VectorSubcoreMesh(core_axis_name="c", subcore_axis_name="s")
    n_workers = 32
    per_w = K // n_workers
    n_chunks = per_w // _CHUNK

    @functools.partial(
        pl.kernel,
        mesh=mesh,
        out_type=jax.ShapeDtypeStruct((K, 128), jnp.float32),
        scratch_types=[
            pltpu.VMEM((_CHUNK,), jnp.int32),
            pltpu.VMEM((_CHUNK, 128), jnp.float32),
            pltpu.SemaphoreType.DMA,
        ],
    )
    def gather_kernel(table_hbm, idx_hbm, out_hbm, idx_v, rows_v, sem):
        wid = jax.lax.axis_index("s") * 2 + jax.lax.axis_index("c")

        @pl.loop(0, n_chunks)
        def _(j):
            base = wid * per_w + j * _CHUNK
            pltpu.sync_copy(idx_hbm.at[pl.ds(base, _CHUNK)], idx_v)
            pltpu.async_copy(table_hbm.at[idx_v], rows_v, sem).wait()
            pltpu.sync_copy(rows_v, out_hbm.at[pl.ds(base, _CHUNK)])

    return gather_kernel(table128, row_idx)


def _extract_sub_body(r_ref, s_ref, c_ref, o_ref):
    rows = r_ref[...]
    sidx = s_ref[...]
    ext = jnp.zeros_like(rows)
    for s in range(8):
        cand = pltpu.roll(rows, 128 - 16 * s, 1) if s else rows
        ext = jnp.where(sidx == s, cand, ext)
    o_ref[...] = ext[:, :16] - c_ref[...]


def _extract_subtract(rows, sub_idx, sub_full):
    K = rows.shape[0]
    blk = 2048
    return pl.pallas_call(
        _extract_sub_body,
        grid=(K // blk,),
        in_specs=[
            pl.BlockSpec((blk, 128), lambda i: (i, 0)),
            pl.BlockSpec((blk, 1), lambda i: (i, 0)),
            pl.BlockSpec((blk, 16), lambda i: (i, 0)),
        ],
        out_specs=pl.BlockSpec((blk, 16), lambda i: (i, 0)),
        out_shape=jax.ShapeDtypeStruct((K, 16), jnp.float32),
    )(rows, sub_idx, sub_full)


def kernel(pc_fts, num_groups, group_size):
    B, N, C = pc_fts.shape
    G = 512
    M = 32
    xyz = pc_fts[..., :3]

    centers = _fps_centers(xyz, G)

    c2 = jnp.sum(centers ** 2, axis=-1)
    x2 = jnp.sum(xyz ** 2, axis=-1)
    cxt = jnp.einsum("bgd,bnd->bgn", centers, xyz)
    d2 = c2[:, :, None] + x2[:, None, :] - 2.0 * cxt
    _, idx = jax.lax.top_k(-d2, M)

    idx_base = jnp.arange(B, dtype=idx.dtype)[:, None, None] * N
    flat_idx = (idx + idx_base).reshape(-1)

    K = B * G * M
    table128 = jnp.pad(pc_fts.reshape(B * N, C), ((0, 0), (0, 16 - C))).reshape(
        B * N // 8, 128
    )
    row_idx = flat_idx // 8
    sub_idx = (flat_idx % 8).reshape(K, 1)
    rows = _sc_gather_rows(table128, row_idx)

    sub_full = jnp.broadcast_to(
        jnp.pad(centers, ((0, 0), (0, 0), (0, 13)))[:, :, None, :], (B, G, M, 16)
    ).reshape(K, 16)
    nb = _extract_subtract(rows, sub_idx, sub_full)
    neighborhoods = nb.reshape(B, G, M, 16)[..., :C]
    return (neighborhoods, centers)


# speedup vs baseline: 9.0773x; 4.8385x over previous
import functools

import jax
import jax.numpy as jnp
from jax.experimental import pallas as pl
from jax.experimental.pallas import tpu as pltpu
from jax.experimental.pallas import tpu_sc as plsc


def _fps_body(x_ref, y_ref, z_ref, cx_ref, cy_ref, cz_ref, dists_ref, niota_ref, *, n_groups):
    B, N = x_ref.shape
    dists_ref[...] = jnp.full((B, N), 1e10, dtype=jnp.float32)
    niota_ref[...] = jax.lax.broadcasted_iota(jnp.int32, (B, N), 1)

    x = x_ref[...]
    y = y_ref[...]
    z = z_ref[...]
    ni = niota_ref[...]

    cx0 = x[:, 0:1]
    cy0 = y[:, 0:1]
    cz0 = z[:, 0:1]

    lane_i = jax.lax.broadcasted_iota(jnp.int32, (B, 128), 1)
    zeros_blk = jnp.zeros((B, 128), dtype=jnp.float32)

    def body(i, carry):
        cx, cy, cz, bx, by, bz = carry
        msk = lane_i == i
        bx = jnp.where(msk, cx, bx)
        by = jnp.where(msk, cy, by)
        bz = jnp.where(msk, cz, bz)
        dx = x - cx
        dy = y - cy
        dz = z - cz
        d = (dx * dx + dz * dz) + dy * dy
        dn = jnp.minimum(dists_ref[...], d)
        dists_ref[...] = dn
        m = jnp.max(dn, axis=1, keepdims=True)
        am = jnp.min(jnp.where(dn == m, ni, jnp.int32(1 << 30)), axis=1, keepdims=True)
        sel = ni == am
        ncx = jnp.max(jnp.where(sel, x, -jnp.inf), axis=1, keepdims=True)
        ncy = jnp.max(jnp.where(sel, y, -jnp.inf), axis=1, keepdims=True)
        ncz = jnp.max(jnp.where(sel, z, -jnp.inf), axis=1, keepdims=True)
        return (ncx, ncy, ncz, bx, by, bz)

    carry = (cx0, cy0, cz0, zeros_blk, zeros_blk, zeros_blk)
    for blk in range(n_groups // 128):
        carry = jax.lax.fori_loop(0, 128, body, carry)
        cx_ref[:, blk * 128:(blk + 1) * 128] = carry[3]
        cy_ref[:, blk * 128:(blk + 1) * 128] = carry[4]
        cz_ref[:, blk * 128:(blk + 1) * 128] = carry[5]


def _fps_centers(xyz, n_groups):
    B, N, _ = xyz.shape
    x = xyz[..., 0]
    y = xyz[..., 1]
    z = xyz[..., 2]
    out_sh = jax.ShapeDtypeStruct((B, n_groups), jnp.float32)
    cx, cy, cz = pl.pallas_call(
        functools.partial(_fps_body, n_groups=n_groups),
        out_shape=(out_sh, out_sh, out_sh),
        scratch_shapes=[
            pltpu.VMEM((B, N), jnp.float32),
            pltpu.VMEM((B, N), jnp.int32),
        ],
    )(x, y, z)
    return jnp.stack([cx, cy, cz], axis=-1)


def _lt(a, b):
    return (a[0] < b[0]) | ((a[0] == b[0]) & (a[1] < b[1]))


def _cas(lst, i, j):
    a, b = lst[i], lst[j]
    m = _lt(a, b)
    lst[i] = (jnp.where(m, a[0], b[0]), jnp.where(m, a[1], b[1]))
    lst[j] = (jnp.where(m, b[0], a[0]), jnp.where(m, b[1], a[1]))


def _minlex(a, b):
    m = _lt(a, b)
    return (jnp.where(m, a[0], b[0]), jnp.where(m, a[1], b[1]))


def _bitonic_sort(lst):
    L = len(lst)
    k = 2
    while k <= L:
        j = k // 2
        while j >= 1:
            for i in range(L):
                l = i ^ j
                if l > i:
                    if (i & k) == 0:
                        _cas(lst, i, l)
                    else:
                        _cas(lst, l, i)
            j //= 2
        k *= 2


def _bitonic_resort32(lst):
    j = 16
    while j >= 1:
        for i in range(32):
            if (i & j) == 0:
                _cas(lst, i, i + j)
        j //= 2


def _merge_top32(a, b):
    c = [_minlex(a[i], b[31 - i]) for i in range(32)]
    _bitonic_resort32(c)
    return c


def _bf16r(v):
    return v.astype(jnp.bfloat16).astype(jnp.float32)


def _knn_body(x_ref, y_ref, z_ref, cx_ref, cy_ref, cz_ref, o_ref):
    N = x_ref.shape[-1]
    R = 8
    xe = jnp.broadcast_to(x_ref[0], (R, N))
    ye = jnp.broadcast_to(y_ref[0], (R, N))
    ze = jnp.broadcast_to(z_ref[0], (R, N))
    xb = _bf16r(xe)
    yb = _bf16r(ye)
    zb = _bf16r(ze)
    cx, cy, cz = cx_ref[0], cy_ref[0], cz_ref[0]
    cbx, cby, cbz = _bf16r(cx), _bf16r(cy), _bf16r(cz)

    x2 = (xe * xe + ze * ze) + ye * ye
    c2 = (cx * cx + cz * cz) + cy * cy
    cxt = (cbx * xb + cby * yb) + cbz * zb
    d2 = (c2 + x2) - 2.0 * cxt

    lane = jax.lax.broadcasted_iota(jnp.int32, (R, 128), 1)
    pairs = [
        (d2[:, t * 128:(t + 1) * 128], lane + t * 128) for t in range(N // 128)
    ]
    lo = pairs[:32]
    hi = pairs[32:]
    _bitonic_sort(lo)
    _bitonic_sort(hi)
    top = _merge_top32(lo, hi)

    for d in (1, 2, 4, 8, 16, 32, 64):
        b = [(pltpu.roll(v, d, 1), pltpu.roll(n, d, 1)) for (v, n) in top]
        top = _merge_top32(top, b)

    lane32 = jax.lax.broadcasted_iota(jnp.int32, (R, 32), 1)
    acc = jnp.zeros((R, 32), jnp.int32)
    for i in range(32):
        col = jnp.broadcast_to(top[i][1][:, 0:1], (R, 32))
        acc = jnp.where(lane32 == i, col, acc)
    o_ref[0] = acc


def _knn_top32(pc_fts, centers):
    B, N, _ = pc_fts.shape
    G = centers.shape[1]
    xyz = pc_fts[..., :3]

    def prep(a):
        return a.reshape(B, 1, N)

    def prepc(a):
        return a.reshape(B * G // 8, 8, 1)

    x = xyz[..., 0]
    y = xyz[..., 1]
    z = xyz[..., 2]

    n_gblk = G // 8
    pt_spec = pl.BlockSpec((1, 1, N), lambda b, j: (b, 0, 0))
    c_spec = pl.BlockSpec((1, 8, 1), lambda b, j: (b * n_gblk + j, 0, 0))
    idx_all = pl.pallas_call(
        _knn_body,
        grid=(B, n_gblk),
        in_specs=[pt_spec] * 3 + [c_spec] * 3,
        out_specs=pl.BlockSpec((1, 8, 32), lambda b, j: (b * n_gblk + j, 0, 0)),
        out_shape=jax.ShapeDtypeStruct((B * G // 8, 8, 32), jnp.int32),
    )(
        prep(x), prep(y), prep(z),
        prepc(centers[..., 0]), prepc(centers[..., 1]), prepc(centers[..., 2]),
    )
    return idx_all.reshape(B, G, 32)


_CHUNK = 128


def _sc_gather_rows(table128, row_idx):
    K = row_idx.shape[0]
    mesh = plsc.VectorSubcoreMesh(core_axis_name="c", subcore_axis_name="s")
    n_workers = 32
    per_w = K // n_workers
    n_chunks = per_w // _CHUNK

    @functools.partial(
        pl.kernel,
        mesh=mesh,
        out_type=jax.ShapeDtypeStruct((K, 128), jnp.float32),
        scratch_types=[
            pltpu.VMEM((_CHUNK,), jnp.int32),
            pltpu.VMEM((_CHUNK, 128), jnp.float32),
            pltpu.SemaphoreType.DMA,
        ],
    )
    def gather_kernel(table_hbm, idx_hbm, out_hbm, idx_v, rows_v, sem):
        wid = jax.lax.axis_index("s") * 2 + jax.lax.axis_index("c")

        @pl.loop(0, n_chunks)
        def _(j):
            base = wid * per_w + j * _CHUNK
            pltpu.sync_copy(idx_hbm.at[pl.ds(base, _CHUNK)], idx_v)
            pltpu.async_copy(table_hbm.at[idx_v], rows_v, sem).wait()
            pltpu.sync_copy(rows_v, out_hbm.at[pl.ds(base, _CHUNK)])

    return gather_kernel(table128, row_idx)


def _extract_sub_body(r_ref, s_ref, c_ref, o_ref):
    rows = r_ref[...]
    sidx = s_ref[...]
    ext = jnp.zeros_like(rows)
    for s in range(8):
        cand = pltpu.roll(rows, 128 - 16 * s, 1) if s else rows
        ext = jnp.where(sidx == s, cand, ext)
    o_ref[...] = ext[:, :16] - c_ref[...]


def _extract_subtract(rows, sub_idx, sub_full):
    K = rows.shape[0]
    blk = 2048
    return pl.pallas_call(
        _extract_sub_body,
        grid=(K // blk,),
        in_specs=[
            pl.BlockSpec((blk, 128), lambda i: (i, 0)),
            pl.BlockSpec((blk, 1), lambda i: (i, 0)),
            pl.BlockSpec((blk, 16), lambda i: (i, 0)),
        ],
        out_specs=pl.BlockSpec((blk, 16), lambda i: (i, 0)),
        out_shape=jax.ShapeDtypeStruct((K, 16), jnp.float32),
    )(rows, sub_idx, sub_full)


def kernel(pc_fts, num_groups, group_size):
    B, N, C = pc_fts.shape
    G = 512
    M = 32
    xyz = pc_fts[..., :3]

    centers = _fps_centers(xyz, G)

    idx = _knn_top32(pc_fts, centers)

    idx_base = jnp.arange(B, dtype=idx.dtype)[:, None, None] * N
    flat_idx = (idx + idx_base).reshape(-1)

    K = B * G * M
    table128 = jnp.pad(pc_fts.reshape(B * N, C), ((0, 0), (0, 16 - C))).reshape(
        B * N // 8, 128
    )
    row_idx = flat_idx // 8
    sub_idx = (flat_idx % 8).reshape(K, 1)
    rows = _sc_gather_rows(table128, row_idx)

    sub_full = jnp.broadcast_to(
        jnp.pad(centers, ((0, 0), (0, 0), (0, 13)))[:, :, None, :], (B, G, M, 16)
    ).reshape(K, 16)
    nb = _extract_subtract(rows, sub_idx, sub_full)
    neighborhoods = nb.reshape(B, G, M, 16)[..., :C]
    return (neighborhoods, centers)
